# Initial kernel scaffold; baseline (speedup 1.0000x reference)
import jax
import jax.numpy as jnp
from jax.experimental import pallas as pl


def kernel(x, edge_index, edge_attr, w1, root1, b1, w2, root2, b2):
    raise NotImplementedError("write your pallas kernel here")


# speedup vs baseline: 7.1582x; 7.1582x over previous
import functools

import jax
import jax.numpy as jnp
from jax import lax
from jax.experimental import pallas as pl
from jax.experimental.pallas import tpu as pltpu
from jax.experimental.pallas import tpu_sc as plsc

N = 10000
E = 320000
F = 128
R = 8

NC = 2
NS = 16
NW = NC * NS
EPW = E // NW
EPT = E // NS
CH = 128
EPTP = 20480
NCH = EPTP // CH
BLK = 8
NBLK = NCH // BLK
HN = N // NC
TRASH = HN
ACCR = 5120
SLAB = 312
TAIL = HN - NS * SLAB
KEYS = R * N
ER = E // F
L = 16


def _prep_body(src_ref, dst_ref, ew_ref, et_ref,
               g_ref, k2_ref, wn_ref, d0_ref, d1_ref):
    ew = ew_ref[...]
    lo = jnp.min(ew)
    hi = jnp.max(ew)
    wn_ref[...] = (ew - lo) / (hi - lo + 1e-08)
    t = et_ref[...].astype(jnp.int32)
    dst = dst_ref[...]
    g_ref[...] = t * N + src_ref[...]
    k2_ref[...] = t * N + dst
    d0_ref[...] = jnp.where(dst < HN, dst, TRASH)
    d1_ref[...] = jnp.where(dst >= HN, dst - HN, TRASH)


def _prep(src, dst, ew, et):
    return pl.pallas_call(
        _prep_body,
        out_shape=(
            jax.ShapeDtypeStruct((ER, F), jnp.int32),
            jax.ShapeDtypeStruct((ER, F), jnp.int32),
            jax.ShapeDtypeStruct((ER, F), jnp.float32),
            jax.ShapeDtypeStruct((ER, F), jnp.int32),
            jax.ShapeDtypeStruct((ER, F), jnp.int32),
        ),
    )(src, dst, ew, et)


def _count(k2w):
    mesh = plsc.VectorSubcoreMesh(core_axis_name="c", subcore_axis_name="s")

    @functools.partial(
        pl.kernel,
        mesh=mesh,
        compiler_params=pltpu.CompilerParams(needs_layout_passes=False),
        out_type=jax.ShapeDtypeStruct((NW, KEYS), jnp.float32),
        scratch_types=[
            pltpu.VMEM((EPW,), jnp.int32),
            pltpu.VMEM((KEYS,), jnp.float32),
        ],
    )
    def k(k2_hbm, out_hbm, keys_v, cnt_v):
        c = lax.axis_index("c")
        s = lax.axis_index("s")
        wid = s * NC + c
        pltpu.sync_copy(k2_hbm.at[wid], keys_v)
        zeros = jnp.zeros((L,), jnp.float32)

        def zbody(i, carry):
            cnt_v[pl.ds(i * L, L)] = zeros
            return carry

        lax.fori_loop(0, KEYS // L, zbody, 0)
        ones = jnp.ones((L,), jnp.float32)

        def body(i, carry):
            kk = keys_v[pl.ds(i * L, L)]
            plsc.addupdate_scatter(cnt_v, [kk], ones)
            return carry

        lax.fori_loop(0, EPW // L, body, 0)
        pltpu.sync_copy(cnt_v, out_hbm.at[wid])

    return k(k2w)


def _csum_body(p_ref, o_ref):
    o_ref[...] = jnp.sum(p_ref[...], axis=0)


def _csum(cntp):
    return pl.pallas_call(
        _csum_body,
        out_shape=jax.ShapeDtypeStruct((KEYS // F, F), jnp.float32),
    )(cntp)


def _acoef(k2w, wnw, cnt):
    mesh = plsc.VectorSubcoreMesh(core_axis_name="c", subcore_axis_name="s")

    @functools.partial(
        pl.kernel,
        mesh=mesh,
        compiler_params=pltpu.CompilerParams(needs_layout_passes=False),
        out_type=jax.ShapeDtypeStruct((NW, EPW), jnp.float32),
        scratch_types=[
            pltpu.VMEM((EPW,), jnp.int32),
            pltpu.VMEM((EPW,), jnp.float32),
            pltpu.VMEM((KEYS,), jnp.float32),
            pltpu.VMEM((EPW,), jnp.float32),
        ],
    )
    def k(k2_hbm, wn_hbm, cnt_hbm, out_hbm, keys_v, wn_v, cnt_v, a_v):
        c = lax.axis_index("c")
        s = lax.axis_index("s")
        wid = s * NC + c
        pltpu.sync_copy(k2_hbm.at[wid], keys_v)
        pltpu.sync_copy(wn_hbm.at[wid], wn_v)
        pltpu.sync_copy(cnt_hbm, cnt_v)

        def body(i, carry):
            sl = pl.ds(i * L, L)
            kk = keys_v[sl]
            cc = plsc.load_gather(cnt_v, [kk])
            a_v[sl] = wn_v[sl] / jnp.maximum(cc, 1.0)
            return carry

        lax.fori_loop(0, EPW // L, body, 0)
        pltpu.sync_copy(a_v, out_hbm.at[wid])

    return k(k2w, wnw, cnt)


def _transform_body(x_ref, w_ref, y_ref):
    y_ref[0] = jnp.dot(x_ref[...], w_ref[0],
                       preferred_element_type=jnp.float32)


def _transform(x, w9):
    bn = 2000
    return pl.pallas_call(
        _transform_body,
        grid=(R + 1, N // bn),
        in_specs=[
            pl.BlockSpec((bn, F), lambda r, i: (i, 0)),
            pl.BlockSpec((1, F, F), lambda r, i: (r, 0, 0)),
        ],
        out_specs=pl.BlockSpec((1, bn, F), lambda r, i: (r, i, 0)),
        out_shape=jax.ShapeDtypeStruct((R + 1, N, F), jnp.float32),
    )(x, w9)


def _make_scatter():
    mesh = plsc.VectorSubcoreMesh(core_axis_name="c", subcore_axis_name="s")

    @functools.partial(
        pl.kernel,
        mesh=mesh,
        compiler_params=pltpu.CompilerParams(needs_layout_passes=False),
        out_type=jax.ShapeDtypeStruct((NC, HN, F), jnp.float32),
        scratch_types=[
            pltpu.VMEM((BLK, CH), jnp.int32),
            pltpu.VMEM((BLK, CH), jnp.int32),
            pltpu.VMEM((BLK, CH), jnp.float32),
            pltpu.VMEM((CH, F), jnp.float32),
            pltpu.VMEM_SHARED((ACCR, F), jnp.float32),
            pltpu.SemaphoreType.DMA,
        ],
    )
    def k(y_hbm, g_hbm, d_hbm, a_hbm, out_hbm,
          g_v, d_v, a_v, rows_v, acc_sh, sem):
        c = lax.axis_index("c")
        s = lax.axis_index("s")

        zeros = jnp.zeros((L,), jnp.float32)

        def zbody(i, carry):
            for kk in range(F // L):
                rows_v[i, pl.ds(kk * L, L)] = zeros
            return carry

        lax.fori_loop(0, CH, zbody, 0)
        base = s * (ACCR // NS)
        pltpu.sync_copy(rows_v, acc_sh.at[pl.ds(base, CH)])
        pltpu.sync_copy(rows_v, acc_sh.at[pl.ds(base + CH, CH)])
        pltpu.sync_copy(rows_v.at[pl.ds(0, ACCR // NS - 2 * CH)],
                        acc_sh.at[pl.ds(base + 2 * CH, ACCR // NS - 2 * CH)])
        plsc.subcore_barrier()

        def block(b, carry):
            bs = pl.ds(b * BLK, BLK)
            pltpu.sync_copy(g_hbm.at[s, bs], g_v)
            pltpu.sync_copy(d_hbm.at[c, s, bs], d_v)
            pltpu.sync_copy(a_hbm.at[s, bs], a_v)

            def chunk(j, carry2):
                pltpu.async_copy(y_hbm.at[g_v.at[j]], rows_v, sem).wait()

                def scale(g, carry3):
                    a16 = a_v[j, pl.ds(g * L, L)]
                    for eo in range(L):
                        e = g * L + eo
                        sv = jnp.full((L,), a16[eo], jnp.float32)
                        for kk in range(F // L):
                            sl = pl.ds(kk * L, L)
                            rows_v[e, sl] = rows_v[e, sl] * sv
                    return carry3

                lax.fori_loop(0, CH // L, scale, 0)
                pltpu.sync_copy(rows_v, acc_sh.at[d_v.at[j]], add=True)
                return carry2

            lax.fori_loop(0, BLK, chunk, 0)
            return carry

        lax.fori_loop(0, NBLK, block, 0)
        plsc.subcore_barrier()
        pltpu.sync_copy(acc_sh.at[pl.ds(s * SLAB, SLAB)],
                        out_hbm.at[c, pl.ds(s * SLAB, SLAB)])

        @pl.when(s == 0)
        def _():
            pltpu.sync_copy(acc_sh.at[pl.ds(NS * SLAB, TAIL)],
                            out_hbm.at[c, pl.ds(NS * SLAB, TAIL)])

    return k


_scatter = _make_scatter()


def _combine_body(y8_ref, b_ref, msg_ref, o_ref, *, relu):
    v = y8_ref[...] + b_ref[...] + msg_ref[...]
    o_ref[...] = jnp.maximum(v, 0.0) if relu else v


def _combine(y8, b, msg, relu):
    return pl.pallas_call(
        functools.partial(_combine_body, relu=relu),
        out_shape=jax.ShapeDtypeStruct((N, F), jnp.float32),
    )(y8, b, msg)


def kernel(x, edge_index, edge_attr, w1, root1, b1, w2, root2, b2):
    src = edge_index[0].reshape(ER, F)
    dst = edge_index[1].reshape(ER, F)
    ew = edge_attr[:, 0].reshape(ER, F)
    et = edge_attr[:, 1].reshape(ER, F)

    g, k2, wn, d0, d1 = _prep(src, dst, ew, et)
    k2w = k2.reshape(NW, EPW)
    cntp = _count(k2w)
    cnt = _csum(cntp.reshape(NW, KEYS // F, F))
    a = _acoef(k2w, wn.reshape(NW, EPW), cnt.reshape(KEYS))

    pad = ((0, 0), (0, EPTP - EPT))
    gq = jnp.pad(g.reshape(NS, EPT), pad).reshape(NS, NCH, CH)
    dq = jnp.stack([
        jnp.pad(d0.reshape(NS, EPT), pad, constant_values=TRASH)
        .reshape(NS, NCH, CH),
        jnp.pad(d1.reshape(NS, EPT), pad, constant_values=TRASH)
        .reshape(NS, NCH, CH),
    ])
    aq = jnp.pad(a.reshape(NS, EPT), pad).reshape(NS, NCH, CH)

    w9a = jnp.concatenate([w1, root1[None]], axis=0)
    y1 = _transform(x, w9a)
    msg1 = _scatter(y1.reshape((R + 1) * N, F), gq, dq, aq)
    h = _combine(y1[R], b1.reshape(1, F), msg1.reshape(N, F), relu=True)

    w9b = jnp.concatenate([w2, root2[None]], axis=0)
    y2 = _transform(h, w9b)
    msg2 = _scatter(y2.reshape((R + 1) * N, F), gq, dq, aq)
    return _combine(y2[R], b2.reshape(1, F), msg2.reshape(N, F), relu=False)


# speedup vs baseline: 8.2089x; 1.1468x over previous
import functools

import jax
import jax.numpy as jnp
from jax import lax
from jax.experimental import pallas as pl
from jax.experimental.pallas import tpu as pltpu
from jax.experimental.pallas import tpu_sc as plsc

N = 10000
E = 320000
F = 128
R = 8

NC = 2
NS = 16
NW = NC * NS
EPW = E // NW
EPT = E // NS
CH = 128
EPTP = 20480
NCH = EPTP // CH
BLK = 32
NBLK = NCH // BLK
HB = BLK // 2
HN = N // NC
TRASH = HN
ACCR = 5120
SLAB = 312
TAIL = HN - NS * SLAB
KEYS = R * N
ER = E // F
L = 16


def _prep_body(src_ref, dst_ref, ew_ref, et_ref,
               g_ref, k2_ref, wn_ref, d0_ref, d1_ref):
    ew = ew_ref[...]
    lo = jnp.min(ew)
    hi = jnp.max(ew)
    wn_ref[...] = (ew - lo) / (hi - lo + 1e-08)
    t = et_ref[...].astype(jnp.int32)
    dst = dst_ref[...]
    g_ref[...] = t * N + src_ref[...]
    k2_ref[...] = t * N + dst
    d0_ref[...] = jnp.where(dst < HN, dst, TRASH)
    d1_ref[...] = jnp.where(dst >= HN, dst - HN, TRASH)


def _prep(src, dst, ew, et):
    return pl.pallas_call(
        _prep_body,
        out_shape=(
            jax.ShapeDtypeStruct((ER, F), jnp.int32),
            jax.ShapeDtypeStruct((ER, F), jnp.int32),
            jax.ShapeDtypeStruct((ER, F), jnp.float32),
            jax.ShapeDtypeStruct((ER, F), jnp.int32),
            jax.ShapeDtypeStruct((ER, F), jnp.int32),
        ),
    )(src, dst, ew, et)


def _count(k2w):
    mesh = plsc.VectorSubcoreMesh(core_axis_name="c", subcore_axis_name="s")

    @functools.partial(
        pl.kernel,
        mesh=mesh,
        compiler_params=pltpu.CompilerParams(needs_layout_passes=False),
        out_type=jax.ShapeDtypeStruct((NW, KEYS), jnp.float32),
        scratch_types=[
            pltpu.VMEM((EPW,), jnp.int32),
            pltpu.VMEM((KEYS,), jnp.float32),
        ],
    )
    def k(k2_hbm, out_hbm, keys_v, cnt_v):
        c = lax.axis_index("c")
        s = lax.axis_index("s")
        wid = s * NC + c
        pltpu.sync_copy(k2_hbm.at[wid], keys_v)
        zeros = jnp.zeros((L,), jnp.float32)

        def zbody(i, carry):
            cnt_v[pl.ds(i * L, L)] = zeros
            return carry

        lax.fori_loop(0, KEYS // L, zbody, 0)
        ones = jnp.ones((L,), jnp.float32)

        def body(i, carry):
            kk = keys_v[pl.ds(i * L, L)]
            plsc.addupdate_scatter(cnt_v, [kk], ones)
            return carry

        lax.fori_loop(0, EPW // L, body, 0)
        pltpu.sync_copy(cnt_v, out_hbm.at[wid])

    return k(k2w)


def _csum_body(p_ref, o_ref):
    o_ref[...] = jnp.sum(p_ref[...], axis=0)


def _csum(cntp):
    return pl.pallas_call(
        _csum_body,
        out_shape=jax.ShapeDtypeStruct((KEYS // F, F), jnp.float32),
    )(cntp)


def _acoef(k2w, wnw, cnt):
    mesh = plsc.VectorSubcoreMesh(core_axis_name="c", subcore_axis_name="s")

    @functools.partial(
        pl.kernel,
        mesh=mesh,
        compiler_params=pltpu.CompilerParams(needs_layout_passes=False),
        out_type=jax.ShapeDtypeStruct((NW, EPW), jnp.float32),
        scratch_types=[
            pltpu.VMEM((EPW,), jnp.int32),
            pltpu.VMEM((EPW,), jnp.float32),
            pltpu.VMEM((KEYS,), jnp.float32),
            pltpu.VMEM((EPW,), jnp.float32),
        ],
    )
    def k(k2_hbm, wn_hbm, cnt_hbm, out_hbm, keys_v, wn_v, cnt_v, a_v):
        c = lax.axis_index("c")
        s = lax.axis_index("s")
        wid = s * NC + c
        pltpu.sync_copy(k2_hbm.at[wid], keys_v)
        pltpu.sync_copy(wn_hbm.at[wid], wn_v)
        pltpu.sync_copy(cnt_hbm, cnt_v)

        def body(i, carry):
            sl = pl.ds(i * L, L)
            kk = keys_v[sl]
            cc = plsc.load_gather(cnt_v, [kk])
            a_v[sl] = wn_v[sl] / jnp.maximum(cc, 1.0)
            return carry

        lax.fori_loop(0, EPW // L, body, 0)
        pltpu.sync_copy(a_v, out_hbm.at[wid])

    return k(k2w, wnw, cnt)


def _transform_body(x_ref, w_ref, y_ref):
    y_ref[0] = jnp.dot(x_ref[...], w_ref[0],
                       preferred_element_type=jnp.float32)


def _transform(x, w9):
    bn = 2000
    return pl.pallas_call(
        _transform_body,
        grid=(R + 1, N // bn),
        in_specs=[
            pl.BlockSpec((bn, F), lambda r, i: (i, 0)),
            pl.BlockSpec((1, F, F), lambda r, i: (r, 0, 0)),
        ],
        out_specs=pl.BlockSpec((1, bn, F), lambda r, i: (r, i, 0)),
        out_shape=jax.ShapeDtypeStruct((R + 1, N, F), jnp.float32),
    )(x, w9)


def _make_scatter():
    mesh = plsc.VectorSubcoreMesh(core_axis_name="c", subcore_axis_name="s")

    @functools.partial(
        pl.kernel,
        mesh=mesh,
        compiler_params=pltpu.CompilerParams(needs_layout_passes=False),
        out_type=jax.ShapeDtypeStruct((NC, HN, F), jnp.float32),
        scratch_types=[
            pltpu.VMEM((BLK, CH), jnp.int32),
            pltpu.VMEM((BLK, CH), jnp.int32),
            pltpu.VMEM((BLK, CH), jnp.float32),
            pltpu.VMEM((CH, F), jnp.float32),
            pltpu.VMEM((CH, F), jnp.float32),
            pltpu.VMEM_SHARED((ACCR, F), jnp.float32),
            pltpu.SemaphoreType.DMA,
            pltpu.SemaphoreType.DMA,
            pltpu.SemaphoreType.DMA,
            pltpu.SemaphoreType.DMA,
        ],
    )
    def k(y_hbm, g_hbm, d_hbm, a_hbm, out_hbm,
          g_v, d_v, a_v, r0, r1, acc_sh, gs0, gs1, ss0, ss1):
        c = lax.axis_index("c")
        s = lax.axis_index("s")

        zeros = jnp.zeros((L,), jnp.float32)

        def zbody(i, carry):
            for kk in range(F // L):
                r0[i, pl.ds(kk * L, L)] = zeros
            return carry

        lax.fori_loop(0, CH, zbody, 0)
        base = s * (ACCR // NS)
        pltpu.sync_copy(r0, acc_sh.at[pl.ds(base, CH)])
        pltpu.sync_copy(r0, acc_sh.at[pl.ds(base + CH, CH)])
        pltpu.sync_copy(r0.at[pl.ds(0, ACCR // NS - 2 * CH)],
                        acc_sh.at[pl.ds(base + 2 * CH, ACCR // NS - 2 * CH)])
        plsc.subcore_barrier()

        def do_scale(rows_v, j):
            def scale(g, carry3):
                a16 = a_v[j, pl.ds(g * L, L)]
                for eo in range(L):
                    e = g * L + eo
                    sv = jnp.full((L,), a16[eo], jnp.float32)
                    for kk in range(F // L):
                        sl = pl.ds(kk * L, L)
                        rows_v[e, sl] = rows_v[e, sl] * sv
                return carry3

            lax.fori_loop(0, CH // L, scale, 0)

        def block(b, carry):
            bs = pl.ds(b * BLK, BLK)
            pltpu.sync_copy(g_hbm.at[s, bs], g_v)
            pltpu.sync_copy(d_hbm.at[c, s, bs], d_v)
            pltpu.sync_copy(a_hbm.at[s, bs], a_v)
            pltpu.async_copy(y_hbm.at[g_v.at[0]], r0, gs0)

            def pair(t, carry2):
                j0 = 2 * t
                @pl.when(t > 0)
                def _():
                    pltpu.make_async_copy(r1, acc_sh.at[d_v.at[j0]], ss1
                                          ).wait()

                pltpu.async_copy(y_hbm.at[g_v.at[j0 + 1]], r1, gs1)
                pltpu.make_async_copy(y_hbm.at[g_v.at[j0]], r0, gs0).wait()
                do_scale(r0, j0)
                pltpu.async_copy(r0, acc_sh.at[d_v.at[j0]], ss0, add=True)

                @pl.when(t < HB - 1)
                def _():
                    pltpu.make_async_copy(r0, acc_sh.at[d_v.at[j0]], ss0
                                          ).wait()
                    pltpu.async_copy(y_hbm.at[g_v.at[j0 + 2]], r0, gs0)

                pltpu.make_async_copy(y_hbm.at[g_v.at[j0 + 1]], r1, gs1
                                      ).wait()
                do_scale(r1, j0 + 1)
                pltpu.async_copy(r1, acc_sh.at[d_v.at[j0 + 1]], ss1, add=True)
                return carry2

            lax.fori_loop(0, HB, pair, 0)
            pltpu.make_async_copy(r0, acc_sh.at[d_v.at[0]], ss0).wait()
            pltpu.make_async_copy(r1, acc_sh.at[d_v.at[0]], ss1).wait()
            return carry

        lax.fori_loop(0, NBLK, block, 0)
        plsc.subcore_barrier()
        pltpu.sync_copy(acc_sh.at[pl.ds(s * SLAB, SLAB)],
                        out_hbm.at[c, pl.ds(s * SLAB, SLAB)])

        @pl.when(s == 0)
        def _():
            pltpu.sync_copy(acc_sh.at[pl.ds(NS * SLAB, TAIL)],
                            out_hbm.at[c, pl.ds(NS * SLAB, TAIL)])

    return k


_scatter = _make_scatter()


def _combine_body(y8_ref, b_ref, msg_ref, o_ref, *, relu):
    v = y8_ref[...] + b_ref[...] + msg_ref[...]
    o_ref[...] = jnp.maximum(v, 0.0) if relu else v


def _combine(y8, b, msg, relu):
    return pl.pallas_call(
        functools.partial(_combine_body, relu=relu),
        out_shape=jax.ShapeDtypeStruct((N, F), jnp.float32),
    )(y8, b, msg)


def kernel(x, edge_index, edge_attr, w1, root1, b1, w2, root2, b2):
    src = edge_index[0].reshape(ER, F)
    dst = edge_index[1].reshape(ER, F)
    ew = edge_attr[:, 0].reshape(ER, F)
    et = edge_attr[:, 1].reshape(ER, F)

    g, k2, wn, d0, d1 = _prep(src, dst, ew, et)
    k2w = k2.reshape(NW, EPW)
    cntp = _count(k2w)
    cnt = _csum(cntp.reshape(NW, KEYS // F, F))
    a = _acoef(k2w, wn.reshape(NW, EPW), cnt.reshape(KEYS))

    pad = ((0, 0), (0, EPTP - EPT))
    gq = jnp.pad(g.reshape(NS, EPT), pad).reshape(NS, NCH, CH)
    dq = jnp.stack([
        jnp.pad(d0.reshape(NS, EPT), pad, constant_values=TRASH)
        .reshape(NS, NCH, CH),
        jnp.pad(d1.reshape(NS, EPT), pad, constant_values=TRASH)
        .reshape(NS, NCH, CH),
    ])
    aq = jnp.pad(a.reshape(NS, EPT), pad).reshape(NS, NCH, CH)

    w9a = jnp.concatenate([w1, root1[None]], axis=0)
    y1 = _transform(x, w9a)
    msg1 = _scatter(y1.reshape((R + 1) * N, F), gq, dq, aq)
    h = _combine(y1[R], b1.reshape(1, F), msg1.reshape(N, F), relu=True)

    w9b = jnp.concatenate([w2, root2[None]], axis=0)
    y2 = _transform(h, w9b)
    msg2 = _scatter(y2.reshape((R + 1) * N, F), gq, dq, aq)
    return _combine(y2[R], b2.reshape(1, F), msg2.reshape(N, F), relu=False)


# speedup vs baseline: 13.1059x; 1.5965x over previous
import functools

import jax
import jax.numpy as jnp
from jax import lax
from jax.experimental import pallas as pl
from jax.experimental.pallas import tpu as pltpu
from jax.experimental.pallas import tpu_sc as plsc

N = 10000
E = 320000
F = 128
R = 8

NC = 2
NS = 16
NW = NC * NS
EPW = E // NW
EPT = E // NS
CH = 128
EPTP = 20480
NCH = EPTP // CH
BLK = 32
NBLK = NCH // BLK
HB = BLK // 2
FH = F // NC
ACCR = 10240
RPT = ACCR // NS
KEYS = R * N
ER = E // F
L = 16


def _prep_body(src_ref, dst_ref, ew_ref, et_ref, g_ref, k2_ref, wn_ref):
    ew = ew_ref[...]
    lo = jnp.min(ew)
    hi = jnp.max(ew)
    wn_ref[...] = (ew - lo) / (hi - lo + 1e-08)
    t = et_ref[...].astype(jnp.int32)
    dst = dst_ref[...]
    g_ref[...] = t * N + src_ref[...]
    k2_ref[...] = t * N + dst


def _prep(src, dst, ew, et):
    return pl.pallas_call(
        _prep_body,
        out_shape=(
            jax.ShapeDtypeStruct((ER, F), jnp.int32),
            jax.ShapeDtypeStruct((ER, F), jnp.int32),
            jax.ShapeDtypeStruct((ER, F), jnp.float32),
        ),
    )(src, dst, ew, et)


def _count(k2w):
    mesh = plsc.VectorSubcoreMesh(core_axis_name="c", subcore_axis_name="s")

    @functools.partial(
        pl.kernel,
        mesh=mesh,
        compiler_params=pltpu.CompilerParams(needs_layout_passes=False),
        out_type=jax.ShapeDtypeStruct((NW, KEYS), jnp.float32),
        scratch_types=[
            pltpu.VMEM((EPW,), jnp.int32),
            pltpu.VMEM((KEYS,), jnp.float32),
        ],
    )
    def k(k2_hbm, out_hbm, keys_v, cnt_v):
        c = lax.axis_index("c")
        s = lax.axis_index("s")
        wid = s * NC + c
        pltpu.sync_copy(k2_hbm.at[wid], keys_v)
        zeros = jnp.zeros((L,), jnp.float32)

        def zbody(i, carry):
            cnt_v[pl.ds(i * L, L)] = zeros
            return carry

        lax.fori_loop(0, KEYS // L, zbody, 0)

        def body(i, carry):
            kk = keys_v[pl.ds(i * L, L)]
            kcnt, klast = plsc.scan_count(kk)
            plsc.addupdate_scatter(cnt_v, [kk], kcnt.astype(jnp.float32),
                                   mask=klast)
            return carry

        lax.fori_loop(0, EPW // L, body, 0)
        pltpu.sync_copy(cnt_v, out_hbm.at[wid])

    return k(k2w)


def _csum_body(p_ref, o_ref):
    o_ref[...] = jnp.sum(p_ref[...], axis=0)


def _csum(cntp):
    return pl.pallas_call(
        _csum_body,
        out_shape=jax.ShapeDtypeStruct((KEYS // F, F), jnp.float32),
    )(cntp)


def _acoef(k2w, wnw, cnt):
    mesh = plsc.VectorSubcoreMesh(core_axis_name="c", subcore_axis_name="s")

    @functools.partial(
        pl.kernel,
        mesh=mesh,
        compiler_params=pltpu.CompilerParams(needs_layout_passes=False),
        out_type=jax.ShapeDtypeStruct((NW, EPW), jnp.float32),
        scratch_types=[
            pltpu.VMEM((EPW,), jnp.int32),
            pltpu.VMEM((EPW,), jnp.float32),
            pltpu.VMEM((KEYS,), jnp.float32),
            pltpu.VMEM((EPW,), jnp.float32),
        ],
    )
    def k(k2_hbm, wn_hbm, cnt_hbm, out_hbm, keys_v, wn_v, cnt_v, a_v):
        c = lax.axis_index("c")
        s = lax.axis_index("s")
        wid = s * NC + c
        pltpu.sync_copy(k2_hbm.at[wid], keys_v)
        pltpu.sync_copy(wn_hbm.at[wid], wn_v)
        pltpu.sync_copy(cnt_hbm, cnt_v)

        def body(i, carry):
            sl = pl.ds(i * L, L)
            kk = keys_v[sl]
            cc = plsc.load_gather(cnt_v, [kk])
            a_v[sl] = wn_v[sl] / jnp.maximum(cc, 1.0)
            return carry

        lax.fori_loop(0, EPW // L, body, 0)
        pltpu.sync_copy(a_v, out_hbm.at[wid])

    return k(k2w, wnw, cnt)


def _transform_body(x_ref, w_ref, y_ref):
    y_ref[0] = jnp.dot(x_ref[...], w_ref[0],
                       preferred_element_type=jnp.float32,
                       precision=lax.Precision.HIGHEST)


def _transform(x, w9):
    bn = 2000
    return pl.pallas_call(
        _transform_body,
        grid=(R + 1, N // bn),
        in_specs=[
            pl.BlockSpec((bn, F), lambda r, i: (i, 0)),
            pl.BlockSpec((1, F, F), lambda r, i: (r, 0, 0)),
        ],
        out_specs=pl.BlockSpec((1, bn, F), lambda r, i: (r, i, 0)),
        out_shape=jax.ShapeDtypeStruct((R + 1, N, F), jnp.float32),
    )(x, w9)


def _make_scatter():
    mesh = plsc.VectorSubcoreMesh(core_axis_name="c", subcore_axis_name="s")

    @functools.partial(
        pl.kernel,
        mesh=mesh,
        compiler_params=pltpu.CompilerParams(
            needs_layout_passes=False, use_tc_tiling_on_sc=False),
        out_type=jax.ShapeDtypeStruct((NC, ACCR, FH), jnp.float32),
        scratch_types=[
            pltpu.VMEM((BLK, CH), jnp.int32),
            pltpu.VMEM((BLK, CH), jnp.int32),
            pltpu.VMEM((BLK, CH), jnp.float32),
            pltpu.VMEM((CH, FH), jnp.float32),
            pltpu.VMEM((CH, FH), jnp.float32),
            pltpu.VMEM_SHARED((ACCR, FH), jnp.float32),
            pltpu.SemaphoreType.DMA,
            pltpu.SemaphoreType.DMA,
            pltpu.SemaphoreType.DMA,
            pltpu.SemaphoreType.DMA,
        ],
    )
    def k(y_hbm, g_hbm, d_hbm, a_hbm, out_hbm,
          g_v, d_v, a_v, r0, r1, acc_sh, gs0, gs1, ss0, ss1):
        c = lax.axis_index("c")
        s = lax.axis_index("s")

        zeros = jnp.zeros((L,), jnp.float32)

        def zbody(i, carry):
            for kk in range(FH // L):
                r0[i, pl.ds(kk * L, L)] = zeros
            return carry

        lax.fori_loop(0, CH, zbody, 0)
        base = s * RPT
        for j in range(RPT // CH):
            pltpu.sync_copy(r0, acc_sh.at[pl.ds(base + j * CH, CH)])
        plsc.subcore_barrier()

        def do_scale(rows_v, j):
            def scale(g, carry3):
                a16 = a_v[j, pl.ds(g * L, L)]
                for eo in range(L):
                    e = g * L + eo
                    sv = jnp.full((L,), a16[eo], jnp.float32)
                    for kk in range(FH // L):
                        sl = pl.ds(kk * L, L)
                        rows_v[e, sl] = rows_v[e, sl] * sv
                return carry3

            lax.fori_loop(0, CH // L, scale, 0)

        def block(b, carry):
            bs = pl.ds(b * BLK, BLK)
            pltpu.sync_copy(g_hbm.at[s, bs], g_v)
            pltpu.sync_copy(d_hbm.at[s, bs], d_v)
            pltpu.sync_copy(a_hbm.at[s, bs], a_v)

            def gfix(i, carry2):
                jr = i // (CH // L)
                sl = pl.ds((i % (CH // L)) * L, L)
                g_v[jr, sl] = g_v[jr, sl] * 2 + c
                return carry2

            lax.fori_loop(0, BLK * (CH // L), gfix, 0)
            pltpu.async_copy(y_hbm.at[g_v.at[0]], r0, gs0)

            def pair(t, carry2):
                j0 = 2 * t
                @pl.when(t > 0)
                def _():
                    pltpu.make_async_copy(r1, acc_sh.at[d_v.at[j0]], ss1
                                          ).wait()

                pltpu.async_copy(y_hbm.at[g_v.at[j0 + 1]], r1, gs1)
                pltpu.make_async_copy(y_hbm.at[g_v.at[j0]], r0, gs0).wait()
                do_scale(r0, j0)
                pltpu.async_copy(r0, acc_sh.at[d_v.at[j0]], ss0, add=True)

                @pl.when(t < HB - 1)
                def _():
                    pltpu.make_async_copy(r0, acc_sh.at[d_v.at[j0]], ss0
                                          ).wait()
                    pltpu.async_copy(y_hbm.at[g_v.at[j0 + 2]], r0, gs0)

                pltpu.make_async_copy(y_hbm.at[g_v.at[j0 + 1]], r1, gs1
                                      ).wait()
                do_scale(r1, j0 + 1)
                pltpu.async_copy(r1, acc_sh.at[d_v.at[j0 + 1]], ss1, add=True)
                return carry2

            lax.fori_loop(0, HB, pair, 0)
            pltpu.make_async_copy(r0, acc_sh.at[d_v.at[0]], ss0).wait()
            pltpu.make_async_copy(r1, acc_sh.at[d_v.at[0]], ss1).wait()
            return carry

        lax.fori_loop(0, NBLK, block, 0)
        plsc.subcore_barrier()
        pltpu.sync_copy(acc_sh.at[pl.ds(base, RPT)],
                        out_hbm.at[c, pl.ds(base, RPT)])

    return k


_scatter = _make_scatter()


def _combine_body(y8_ref, b_ref, msg_ref, o_ref, *, relu):
    v = y8_ref[...] + b_ref[...] + msg_ref[...]
    o_ref[...] = jnp.maximum(v, 0.0) if relu else v


def _combine(y8, b, msg, relu):
    return pl.pallas_call(
        functools.partial(_combine_body, relu=relu),
        out_shape=jax.ShapeDtypeStruct((N, F), jnp.float32),
    )(y8, b, msg)


def kernel(x, edge_index, edge_attr, w1, root1, b1, w2, root2, b2):
    src = edge_index[0].reshape(ER, F)
    dst = edge_index[1].reshape(ER, F)
    ew = edge_attr[:, 0].reshape(ER, F)
    et = edge_attr[:, 1].reshape(ER, F)

    g, k2, wn = _prep(src, dst, ew, et)
    k2w = k2.reshape(NW, EPW)
    cntp = _count(k2w)
    cnt = _csum(cntp.reshape(NW, KEYS // F, F))
    a = _acoef(k2w, wn.reshape(NW, EPW), cnt.reshape(KEYS))

    pad = ((0, 0), (0, EPTP - EPT))
    gq = jnp.pad(g.reshape(NS, EPT), pad).reshape(NS, NCH, CH)
    dq = jnp.pad(edge_index[1].reshape(NS, EPT), pad).reshape(NS, NCH, CH)
    aq = jnp.pad(a.reshape(NS, EPT), pad).reshape(NS, NCH, CH)

    def _layer(xin, w, root):
        y = _transform(xin, jnp.concatenate([w, root[None]], axis=0))
        m = _scatter(y.reshape((R + 1) * N * NC, FH), gq, dq, aq)
        msg = jnp.concatenate([m[0, :N], m[1, :N]], axis=1)
        return y[R], msg

    y8a, msg1 = _layer(x, w1, root1)
    h = _combine(y8a, b1.reshape(1, F), msg1, relu=True)
    y8b, msg2 = _layer(h, w2, root2)
    return _combine(y8b, b2.reshape(1, F), msg2, relu=False)
